# Initial kernel scaffold; baseline (speedup 1.0000x reference)
import jax
import jax.numpy as jnp
from jax.experimental import pallas as pl


def kernel(x, params, edge_index, batch):
    raise NotImplementedError("write your pallas kernel here")


# speedup vs baseline: 3.7121x; 3.7121x over previous
import functools

import jax
import jax.numpy as jnp
from jax import lax
from jax.experimental import pallas as pl
from jax.experimental.pallas import tpu as pltpu
from jax.experimental.pallas import tpu_sc as plsc

F32 = jnp.float32

_NCORES = 2
_NSUB = 16
_NW = _NCORES * _NSUB

_EB = 128


def _bn(t, w, b, n_rows, eps=1e-5):
    mu = jnp.mean(t, axis=0, keepdims=True)
    var = jnp.mean((t - mu) ** 2, axis=0, keepdims=True)
    return (t - mu) * lax.rsqrt(var + eps) * w + b


def _pre_body(x_ref, bw_ref, bb_ref, w_ref, b_ref, o_ref):
    x = x_ref[...]
    h = _bn(x, bw_ref[...], bb_ref[...], x.shape[0])
    h = jnp.maximum(jnp.dot(h, w_ref[...], preferred_element_type=F32)
                    + b_ref[...], 0.0)
    o_ref[...] = h


def _pre(x, bw, bb, w, b):
    return pl.pallas_call(
        _pre_body,
        out_shape=jax.ShapeDtypeStruct(x.shape, F32),
    )(x, bw, bb, w, b)


def _gin_body(h_ref, p_ref, w1_ref, b1_ref, bw_ref, bb_ref, w2_ref, b2_ref,
              o_ref):
    n = h_ref.shape[0]
    agg = h_ref[...] + p_ref[0, :n] + p_ref[1, :n]
    t = jnp.dot(agg, w1_ref[...], preferred_element_type=F32) + b1_ref[...]
    t = _bn(t, bw_ref[...], bb_ref[...], t.shape[0])
    t = jnp.maximum(t, 0.0)
    t = jnp.dot(t, w2_ref[...], preferred_element_type=F32) + b2_ref[...]
    o_ref[...] = jnp.maximum(t, 0.0)


def _gin(h, parts, w1, b1, bw, bb, w2, b2):
    return pl.pallas_call(
        _gin_body,
        out_shape=jax.ShapeDtypeStruct(h.shape, F32),
    )(h, parts, w1, b1, bw, bb, w2, b2)


def _head_body(h_ref, batch_ref, fbw_ref, fbb_ref, fw_ref, fb_ref,
               hbw_ref, hbb_ref, cw_ref, cb_ref, o_ref, *, num_graphs):
    h = h_ref[...]
    n = h.shape[0]
    onehot = (batch_ref[...] ==
              lax.broadcasted_iota(jnp.int32, (n, num_graphs), 1)).astype(F32)
    pooled = lax.dot_general(onehot, h, (((0,), (0,)), ((), ())),
                             preferred_element_type=F32)
    z = _bn(pooled, fbw_ref[...], fbb_ref[...], num_graphs)
    z = jnp.maximum(jnp.dot(z, fw_ref[...], preferred_element_type=F32)
                    + fb_ref[...], 0.0)
    z = _bn(z, hbw_ref[...], hbb_ref[...], num_graphs)
    z = jnp.dot(z, cw_ref[...], preferred_element_type=F32) + cb_ref[...]
    m = jnp.max(z, axis=-1, keepdims=True)
    lse = jnp.log(jnp.sum(jnp.exp(z - m), axis=-1, keepdims=True)) + m
    o_ref[...] = z - lse


def _head(h, batch2d, num_graphs, fbw, fbb, fw, fb, hbw, hbb, cw, cb):
    nc = cw.shape[1]
    return pl.pallas_call(
        functools.partial(_head_body, num_graphs=num_graphs),
        out_shape=jax.ShapeDtypeStruct((num_graphs, nc), F32),
    )(h, batch2d, fbw, fbb, fw, fb, hbw, hbb, cw, cb)


def _make_agg(n_nodes, nfeat, e_pad):
    epw = e_pad // _NW
    nblk = epw // _EB
    nacc = ((n_nodes + _NSUB * _EB - 1) // (_NSUB * _EB)) * (_NSUB * _EB)
    rows_per_sub = nacc // _NSUB

    mesh = plsc.VectorSubcoreMesh(core_axis_name="c", subcore_axis_name="s",
                                  num_cores=_NCORES, num_subcores=_NSUB)

    @functools.partial(
        pl.kernel,
        mesh=mesh,
        out_type=jax.ShapeDtypeStruct((_NCORES, nacc, nfeat), F32),
        scratch_types=[
            pltpu.VMEM((_EB,), jnp.int32),
            pltpu.VMEM((_EB,), jnp.int32),
            pltpu.VMEM((_EB, nfeat), F32),
            pltpu.VMEM_SHARED((nacc, nfeat), F32),
            pltpu.SemaphoreType.DMA,
        ],
    )
    def agg(h_hbm, src_hbm, dst_hbm, out_hbm, srcv, dstv, rows, acc, sem):
        c = lax.axis_index("c")
        s = lax.axis_index("s")
        wid = s * _NCORES + c

        def zrow(i, carry):
            for j in range(nfeat // 16):
                rows[i, pl.ds(16 * j, 16)] = jnp.zeros((16,), F32)
            return carry
        lax.fori_loop(0, _EB, zrow, 0)
        for j in range(rows_per_sub // _EB):
            pltpu.sync_copy(rows, acc.at[pl.ds(s * rows_per_sub + j * _EB,
                                               _EB)])
        plsc.subcore_barrier()

        base = wid * epw

        def block(b, carry):
            off = base + b * _EB
            pltpu.sync_copy(src_hbm.at[pl.ds(off, _EB)], srcv)
            pltpu.sync_copy(dst_hbm.at[pl.ds(off, _EB)], dstv)
            pltpu.async_copy(h_hbm.at[srcv], rows, sem).wait()
            pltpu.sync_copy(rows, acc.at[dstv], add=True)
            return carry
        lax.fori_loop(0, nblk, block, 0)
        plsc.subcore_barrier()

        r0 = s * rows_per_sub
        pltpu.sync_copy(acc.at[pl.ds(r0, rows_per_sub)],
                        out_hbm.at[c, pl.ds(r0, rows_per_sub)])

    return agg


def kernel(x, params, edge_index, batch):
    n, nf = x.shape
    hid = params['conv_W'].shape[1]
    g = 128
    e = edge_index.shape[1]

    e_pad = ((e + _NW * _EB - 1) // (_NW * _EB)) * (_NW * _EB)
    pad = e_pad - e
    src = jnp.concatenate([edge_index[0], jnp.zeros((pad,), jnp.int32)])
    dst = jnp.concatenate([edge_index[1], jnp.full((pad,), n, jnp.int32)])

    r = lambda v: v.reshape(1, -1)
    h = _pre(x, r(params['bn_feat_w']), r(params['bn_feat_b']),
             params['conv_W'], r(params['conv_b']))

    agg = _make_agg(n, hid, e_pad)
    for layer in params['gin']:
        parts = agg(h, src, dst)
        h = _gin(h, parts, layer['W1'], r(layer['b1']),
                 r(layer['bn_w']), r(layer['bn_b']),
                 layer['W2'], r(layer['b2']))

    out = _head(h, batch.reshape(-1, 1), g,
                r(params['fc_bn_w']), r(params['fc_bn_b']),
                params['fc_W'], r(params['fc_b']),
                r(params['bn_hid_w']), r(params['bn_hid_b']),
                params['cls_W'], r(params['cls_b']))
    return out


# speedup vs baseline: 4.9945x; 1.3455x over previous
import functools

import jax
import jax.numpy as jnp
from jax import lax
from jax.experimental import pallas as pl
from jax.experimental.pallas import tpu as pltpu
from jax.experimental.pallas import tpu_sc as plsc

F32 = jnp.float32

_NCORES = 2
_NSUB = 16
_NW = _NCORES * _NSUB

_EB = 128


def _bn(t, w, b, n_rows, eps=1e-5):
    mu = jnp.mean(t, axis=0, keepdims=True)
    var = jnp.mean((t - mu) ** 2, axis=0, keepdims=True)
    return (t - mu) * lax.rsqrt(var + eps) * w + b


def _pre_body(x_ref, bw_ref, bb_ref, w_ref, b_ref, o_ref):
    x = x_ref[...]
    h = _bn(x, bw_ref[...], bb_ref[...], x.shape[0])
    h = jnp.maximum(jnp.dot(h, w_ref[...], preferred_element_type=F32)
                    + b_ref[...], 0.0)
    o_ref[...] = h


def _pre(x, bw, bb, w, b):
    return pl.pallas_call(
        _pre_body,
        out_shape=jax.ShapeDtypeStruct(x.shape, F32),
    )(x, bw, bb, w, b)


def _gin_body(h_ref, p_ref, w1_ref, b1_ref, bw_ref, bb_ref, w2_ref, b2_ref,
              o_ref):
    n = h_ref.shape[0]
    agg = h_ref[...] + p_ref[0, :n] + p_ref[1, :n]
    t = jnp.dot(agg, w1_ref[...], preferred_element_type=F32) + b1_ref[...]
    t = _bn(t, bw_ref[...], bb_ref[...], t.shape[0])
    t = jnp.maximum(t, 0.0)
    t = jnp.dot(t, w2_ref[...], preferred_element_type=F32) + b2_ref[...]
    o_ref[...] = jnp.maximum(t, 0.0)


def _gin(h, parts, w1, b1, bw, bb, w2, b2):
    return pl.pallas_call(
        _gin_body,
        out_shape=jax.ShapeDtypeStruct(h.shape, F32),
    )(h, parts, w1, b1, bw, bb, w2, b2)


def _head_body(h_ref, batch_ref, fbw_ref, fbb_ref, fw_ref, fb_ref,
               hbw_ref, hbb_ref, cw_ref, cb_ref, o_ref, *, num_graphs):
    h = h_ref[...]
    n = h.shape[0]
    onehot = (batch_ref[...] ==
              lax.broadcasted_iota(jnp.int32, (n, num_graphs), 1)).astype(F32)
    pooled = lax.dot_general(onehot, h, (((0,), (0,)), ((), ())),
                             preferred_element_type=F32)
    z = _bn(pooled, fbw_ref[...], fbb_ref[...], num_graphs)
    z = jnp.maximum(jnp.dot(z, fw_ref[...], preferred_element_type=F32)
                    + fb_ref[...], 0.0)
    z = _bn(z, hbw_ref[...], hbb_ref[...], num_graphs)
    z = jnp.dot(z, cw_ref[...], preferred_element_type=F32) + cb_ref[...]
    m = jnp.max(z, axis=-1, keepdims=True)
    lse = jnp.log(jnp.sum(jnp.exp(z - m), axis=-1, keepdims=True)) + m
    o_ref[...] = z - lse


def _head(h, batch2d, num_graphs, fbw, fbb, fw, fb, hbw, hbb, cw, cb):
    nc = cw.shape[1]
    return pl.pallas_call(
        functools.partial(_head_body, num_graphs=num_graphs),
        out_shape=jax.ShapeDtypeStruct((num_graphs, nc), F32),
    )(h, batch2d, fbw, fbb, fw, fb, hbw, hbb, cw, cb)


def _make_agg(n_nodes, nfeat, e_pad):
    epw = e_pad // _NW
    nblk = epw // _EB
    nacc = ((n_nodes + 8 * _NSUB) // (8 * _NSUB)) * (8 * _NSUB)
    rows_per_sub = nacc // _NSUB

    mesh = plsc.VectorSubcoreMesh(core_axis_name="c", subcore_axis_name="s",
                                  num_cores=_NCORES, num_subcores=_NSUB)

    @functools.partial(
        pl.kernel,
        mesh=mesh,
        out_type=jax.ShapeDtypeStruct((_NCORES, nacc, nfeat), F32),
        scratch_types=[
            pltpu.VMEM((nblk, _EB), jnp.int32),
            pltpu.VMEM((2, _EB), jnp.int32),
            pltpu.VMEM((2 * _EB, nfeat), F32),
            pltpu.VMEM_SHARED((nacc, nfeat), F32),
            pltpu.SemaphoreType.DMA,
            pltpu.SemaphoreType.DMA,
            pltpu.SemaphoreType.DMA,
            pltpu.SemaphoreType.DMA,
        ],
    )
    def agg(h_hbm, src_hbm, dst_hbm, out_hbm, srcv, dstv, rows, acc,
            sem_si, sem_di, sem_g, sem_s):
        c = lax.axis_index("c")
        s = lax.axis_index("s")
        wid = s * _NCORES + c

        pltpu.async_copy(src_hbm.at[wid], srcv, sem_si)
        pltpu.async_copy(dst_hbm.at[wid, 0], dstv.at[0], sem_di)
        if nblk > 1:
            pltpu.async_copy(dst_hbm.at[wid, 1], dstv.at[1], sem_di)

        def zrow(i, carry):
            for j in range(nfeat // 16):
                rows[i, pl.ds(16 * j, 16)] = jnp.zeros((16,), F32)
            return carry
        lax.fori_loop(0, _EB, zrow, 0)
        for j in range((rows_per_sub + _EB - 1) // _EB):
            zr = min(_EB, rows_per_sub - j * _EB)
            pltpu.sync_copy(
                rows.at[pl.ds(0, zr)],
                acc.at[pl.ds(s * rows_per_sub + j * _EB, zr)])
        pltpu.make_async_copy(src_hbm.at[wid], srcv, sem_si).wait()
        plsc.subcore_barrier()

        def buf(b):
            return rows.at[pl.ds(lax.rem(b, 2) * _EB, _EB)]

        pltpu.async_copy(h_hbm.at[srcv.at[0]], buf(0), sem_g)

        def block(b, carry):
            @pl.when(b > 0)
            def _():
                pltpu.make_async_copy(buf(b - 1), acc.at[dstv.at[
                    lax.rem(b - 1, 2)]], sem_s).wait()

            @pl.when(jnp.logical_and(b > 0, b + 1 < nblk))
            def _():
                pltpu.async_copy(dst_hbm.at[wid, b + 1],
                                 dstv.at[lax.rem(b + 1, 2)], sem_di)

            @pl.when(b + 1 < nblk)
            def _():
                pltpu.async_copy(h_hbm.at[srcv.at[b + 1]], buf(b + 1), sem_g)

            pltpu.make_async_copy(h_hbm.at[srcv.at[b]], buf(b), sem_g).wait()
            pltpu.make_async_copy(dst_hbm.at[wid, b],
                                  dstv.at[lax.rem(b, 2)], sem_di).wait()
            pltpu.async_copy(buf(b), acc.at[dstv.at[lax.rem(b, 2)]], sem_s,
                             add=True)
            return carry
        lax.fori_loop(0, nblk, block, 0)
        pltpu.make_async_copy(buf(nblk - 1),
                              acc.at[dstv.at[lax.rem(nblk - 1, 2)]],
                              sem_s).wait()
        plsc.subcore_barrier()

        r0 = s * rows_per_sub
        pltpu.sync_copy(acc.at[pl.ds(r0, rows_per_sub)],
                        out_hbm.at[c, pl.ds(r0, rows_per_sub)])

    return agg


def kernel(x, params, edge_index, batch):
    n, nf = x.shape
    hid = params['conv_W'].shape[1]
    g = 128
    e = edge_index.shape[1]

    e_pad = ((e + _NW * _EB - 1) // (_NW * _EB)) * (_NW * _EB)
    pad = e_pad - e
    nblk = e_pad // (_NW * _EB)
    src = jnp.concatenate([edge_index[0], jnp.zeros((pad,), jnp.int32)])
    dst = jnp.concatenate([edge_index[1], jnp.full((pad,), n, jnp.int32)])
    src = src.reshape(_NW, nblk, _EB)
    dst = dst.reshape(_NW, nblk, _EB)

    r = lambda v: v.reshape(1, -1)
    h = _pre(x, r(params['bn_feat_w']), r(params['bn_feat_b']),
             params['conv_W'], r(params['conv_b']))

    agg = _make_agg(n, hid, e_pad)
    for layer in params['gin']:
        parts = agg(h, src, dst)
        h = _gin(h, parts, layer['W1'], r(layer['b1']),
                 r(layer['bn_w']), r(layer['bn_b']),
                 layer['W2'], r(layer['b2']))

    out = _head(h, batch.reshape(-1, 1), g,
                r(params['fc_bn_w']), r(params['fc_bn_b']),
                params['fc_W'], r(params['fc_b']),
                r(params['bn_hid_w']), r(params['bn_hid_b']),
                params['cls_W'], r(params['cls_b']))
    return out


# speedup vs baseline: 7.1544x; 1.4324x over previous
import functools

import jax
import jax.numpy as jnp
from jax import lax
from jax.experimental import pallas as pl
from jax.experimental.pallas import tpu as pltpu
from jax.experimental.pallas import tpu_sc as plsc

F32 = jnp.float32

_NCORES = 2
_NSUB = 16
_NW = _NCORES * _NSUB

_EB = 96
_NBUF = 3


def _bn(t, w, b, n_rows, eps=1e-5):
    mu = jnp.mean(t, axis=0, keepdims=True)
    var = jnp.mean((t - mu) ** 2, axis=0, keepdims=True)
    return (t - mu) * lax.rsqrt(var + eps) * w + b


def _pre_body(x_ref, bw_ref, bb_ref, w_ref, b_ref, o_ref):
    x = x_ref[...]
    h = _bn(x, bw_ref[...], bb_ref[...], x.shape[0])
    h = jnp.maximum(jnp.dot(h, w_ref[...], preferred_element_type=F32)
                    + b_ref[...], 0.0)
    o_ref[...] = h


def _pre(x, bw, bb, w, b):
    return pl.pallas_call(
        _pre_body,
        out_shape=jax.ShapeDtypeStruct(x.shape, F32),
    )(x, bw, bb, w, b)


def _gin_body(h_ref, p_ref, w1_ref, b1_ref, bw_ref, bb_ref, w2_ref, b2_ref,
              o_ref):
    n = h_ref.shape[0]
    agg = h_ref[...] + p_ref[0, :n] + p_ref[1, :n]
    t = jnp.dot(agg, w1_ref[...], preferred_element_type=F32) + b1_ref[...]
    t = _bn(t, bw_ref[...], bb_ref[...], t.shape[0])
    t = jnp.maximum(t, 0.0)
    t = jnp.dot(t, w2_ref[...], preferred_element_type=F32) + b2_ref[...]
    o_ref[...] = jnp.maximum(t, 0.0)


def _gin(h, parts, w1, b1, bw, bb, w2, b2):
    return pl.pallas_call(
        _gin_body,
        out_shape=jax.ShapeDtypeStruct(h.shape, F32),
    )(h, parts, w1, b1, bw, bb, w2, b2)


def _head_body(h_ref, batch_ref, fbw_ref, fbb_ref, fw_ref, fb_ref,
               hbw_ref, hbb_ref, cw_ref, cb_ref, o_ref, *, num_graphs):
    h = h_ref[...]
    n = h.shape[0]
    onehot = (batch_ref[...] ==
              lax.broadcasted_iota(jnp.int32, (n, num_graphs), 1)).astype(F32)
    pooled = lax.dot_general(onehot, h, (((0,), (0,)), ((), ())),
                             preferred_element_type=F32)
    z = _bn(pooled, fbw_ref[...], fbb_ref[...], num_graphs)
    z = jnp.maximum(jnp.dot(z, fw_ref[...], preferred_element_type=F32)
                    + fb_ref[...], 0.0)
    z = _bn(z, hbw_ref[...], hbb_ref[...], num_graphs)
    z = jnp.dot(z, cw_ref[...], preferred_element_type=F32) + cb_ref[...]
    m = jnp.max(z, axis=-1, keepdims=True)
    lse = jnp.log(jnp.sum(jnp.exp(z - m), axis=-1, keepdims=True)) + m
    o_ref[...] = z - lse


def _head(h, batch2d, num_graphs, fbw, fbb, fw, fb, hbw, hbb, cw, cb):
    nc = cw.shape[1]
    return pl.pallas_call(
        functools.partial(_head_body, num_graphs=num_graphs),
        out_shape=jax.ShapeDtypeStruct((num_graphs, nc), F32),
    )(h, batch2d, fbw, fbb, fw, fb, hbw, hbb, cw, cb)


def _make_agg(n_nodes, nfeat, e_pad):
    epw = e_pad // _NW
    nblk = epw // _EB
    nacc = ((n_nodes + 8) // 8) * 8
    sub_rows = ((nacc // _NSUB + 7) // 8) * 8
    last_rows = nacc - sub_rows * (_NSUB - 1)

    mesh = plsc.VectorSubcoreMesh(core_axis_name="c", subcore_axis_name="s",
                                  num_cores=_NCORES, num_subcores=_NSUB)

    @functools.partial(
        pl.kernel,
        mesh=mesh,
        out_type=jax.ShapeDtypeStruct((_NCORES, nacc, nfeat), F32),
        scratch_types=[
            pltpu.VMEM((epw,), jnp.int32),
            pltpu.VMEM((_NBUF, _EB), jnp.int32),
            pltpu.VMEM((_NBUF * _EB, nfeat), F32),
            pltpu.VMEM_SHARED((nacc, nfeat), F32),
            pltpu.SemaphoreType.DMA,
            pltpu.SemaphoreType.DMA,
            pltpu.SemaphoreType.DMA,
            pltpu.SemaphoreType.DMA,
        ],
    )
    def agg(h_hbm, src_hbm, dst_hbm, out_hbm, srcv, dstv, rows, acc,
            sem_si, sem_di, sem_g, sem_s):
        c = lax.axis_index("c")
        s = lax.axis_index("s")
        wid = s * _NCORES + c

        pltpu.async_copy(src_hbm.at[wid], srcv, sem_si)
        for j in range(min(_NBUF - 1, nblk)):
            pltpu.async_copy(dst_hbm.at[wid, j], dstv.at[j], sem_di)

        def zrow(i, carry):
            for j in range(nfeat // 16):
                rows[i, pl.ds(16 * j, 16)] = jnp.zeros((16,), F32)
            return carry
        lax.fori_loop(0, _EB, zrow, 0)
        r0 = s * sub_rows

        def zero_span(nrows):
            for j in range((nrows + _EB - 1) // _EB):
                zr = min(_EB, nrows - j * _EB)
                pltpu.sync_copy(rows.at[pl.ds(0, zr)],
                                acc.at[pl.ds(r0 + j * _EB, zr)])

        @pl.when(s < _NSUB - 1)
        def _():
            zero_span(sub_rows)

        @pl.when(s == _NSUB - 1)
        def _():
            zero_span(last_rows)

        pltpu.make_async_copy(src_hbm.at[wid], srcv, sem_si).wait()
        plsc.subcore_barrier()

        def buf(b):
            return rows.at[pl.ds(lax.rem(b, _NBUF) * _EB, _EB)]

        def dslot(b):
            return dstv.at[lax.rem(b, _NBUF)]

        for j in range(min(_NBUF - 1, nblk)):
            pltpu.async_copy(h_hbm.at[srcv.at[pl.ds(j * _EB, _EB)]], buf(j), sem_g)

        def block(b, carry):
            @pl.when(b > 0)
            def _():
                pltpu.make_async_copy(buf(b - 1), acc.at[dslot(b - 1)],
                                      sem_s).wait()

            nxt = b + _NBUF - 1

            @pl.when(nxt < nblk)
            def _():
                pltpu.async_copy(h_hbm.at[srcv.at[pl.ds(nxt * _EB, _EB)]],
                                 buf(nxt), sem_g)
                pltpu.async_copy(dst_hbm.at[wid, nxt], dslot(nxt), sem_di)

            pltpu.make_async_copy(h_hbm.at[srcv.at[pl.ds(b * _EB, _EB)]],
                                  buf(b), sem_g).wait()
            pltpu.make_async_copy(dst_hbm.at[wid, b], dslot(b),
                                  sem_di).wait()
            pltpu.async_copy(buf(b), acc.at[dslot(b)], sem_s, add=True)
            return carry
        lax.fori_loop(0, nblk, block, 0)
        pltpu.make_async_copy(buf(nblk - 1), acc.at[dslot(nblk - 1)],
                              sem_s).wait()
        plsc.subcore_barrier()

        @pl.when(s < _NSUB - 1)
        def _():
            pltpu.sync_copy(acc.at[pl.ds(r0, sub_rows)],
                            out_hbm.at[c, pl.ds(r0, sub_rows)])

        @pl.when(s == _NSUB - 1)
        def _():
            pltpu.sync_copy(acc.at[pl.ds(r0, last_rows)],
                            out_hbm.at[c, pl.ds(r0, last_rows)])

    return agg


def kernel(x, params, edge_index, batch):
    n, nf = x.shape
    hid = params['conv_W'].shape[1]
    g = 128
    e = edge_index.shape[1]

    e_pad = ((e + _NW * _EB - 1) // (_NW * _EB)) * (_NW * _EB)
    pad = e_pad - e
    nblk = e_pad // (_NW * _EB)
    src = jnp.concatenate([edge_index[0], jnp.zeros((pad,), jnp.int32)])
    dst = jnp.concatenate([edge_index[1], jnp.full((pad,), n, jnp.int32)])
    src = src.reshape(_NW, nblk * _EB)
    dst = dst.reshape(_NW, nblk, _EB)

    r = lambda v: v.reshape(1, -1)
    h = _pre(x, r(params['bn_feat_w']), r(params['bn_feat_b']),
             params['conv_W'], r(params['conv_b']))

    agg = _make_agg(n, hid, e_pad)
    for layer in params['gin']:
        parts = agg(h, src, dst)
        h = _gin(h, parts, layer['W1'], r(layer['b1']),
                 r(layer['bn_w']), r(layer['bn_b']),
                 layer['W2'], r(layer['b2']))

    out = _head(h, batch.reshape(-1, 1), g,
                r(params['fc_bn_w']), r(params['fc_bn_b']),
                params['fc_W'], r(params['fc_b']),
                r(params['bn_hid_w']), r(params['bn_hid_b']),
                params['cls_W'], r(params['cls_b']))
    return out


# speedup vs baseline: 8.9472x; 1.2506x over previous
import functools

import jax
import jax.numpy as jnp
from jax import lax
from jax.experimental import pallas as pl
from jax.experimental.pallas import tpu as pltpu
from jax.experimental.pallas import tpu_sc as plsc

F32 = jnp.float32

_NCORES = 2
_NSUB = 16
_NW = _NCORES * _NSUB

_EB = 88
_NBUF = 4
_NIDX = 6


def _bn(t, w, b, n_rows, eps=1e-5):
    mu = jnp.mean(t, axis=0, keepdims=True)
    var = jnp.mean((t - mu) ** 2, axis=0, keepdims=True)
    return (t - mu) * lax.rsqrt(var + eps) * w + b


def _pre_body(x_ref, bw_ref, bb_ref, w_ref, b_ref, o_ref):
    x = x_ref[...]
    h = _bn(x, bw_ref[...], bb_ref[...], x.shape[0])
    h = jnp.maximum(jnp.dot(h, w_ref[...], preferred_element_type=F32)
                    + b_ref[...], 0.0)
    o_ref[...] = h


def _pre(x, bw, bb, w, b):
    return pl.pallas_call(
        _pre_body,
        out_shape=jax.ShapeDtypeStruct(x.shape, F32),
    )(x, bw, bb, w, b)


def _gin_body(h_ref, p_ref, w1_ref, b1_ref, bw_ref, bb_ref, w2_ref, b2_ref,
              o_ref):
    n = h_ref.shape[0]
    agg = h_ref[...] + p_ref[0, :n] + p_ref[1, :n]
    t = jnp.dot(agg, w1_ref[...], preferred_element_type=F32) + b1_ref[...]
    t = _bn(t, bw_ref[...], bb_ref[...], t.shape[0])
    t = jnp.maximum(t, 0.0)
    t = jnp.dot(t, w2_ref[...], preferred_element_type=F32) + b2_ref[...]
    o_ref[...] = jnp.maximum(t, 0.0)


def _gin(h, parts, w1, b1, bw, bb, w2, b2):
    return pl.pallas_call(
        _gin_body,
        out_shape=jax.ShapeDtypeStruct(h.shape, F32),
    )(h, parts, w1, b1, bw, bb, w2, b2)


def _head_body(h_ref, batch_ref, fbw_ref, fbb_ref, fw_ref, fb_ref,
               hbw_ref, hbb_ref, cw_ref, cb_ref, o_ref, *, num_graphs):
    h = h_ref[...]
    n = h.shape[0]
    onehot = (batch_ref[...] ==
              lax.broadcasted_iota(jnp.int32, (n, num_graphs), 1)).astype(F32)
    pooled = lax.dot_general(onehot, h, (((0,), (0,)), ((), ())),
                             preferred_element_type=F32)
    z = _bn(pooled, fbw_ref[...], fbb_ref[...], num_graphs)
    z = jnp.maximum(jnp.dot(z, fw_ref[...], preferred_element_type=F32)
                    + fb_ref[...], 0.0)
    z = _bn(z, hbw_ref[...], hbb_ref[...], num_graphs)
    z = jnp.dot(z, cw_ref[...], preferred_element_type=F32) + cb_ref[...]
    m = jnp.max(z, axis=-1, keepdims=True)
    lse = jnp.log(jnp.sum(jnp.exp(z - m), axis=-1, keepdims=True)) + m
    o_ref[...] = z - lse


def _head(h, batch2d, num_graphs, fbw, fbb, fw, fb, hbw, hbb, cw, cb):
    nc = cw.shape[1]
    return pl.pallas_call(
        functools.partial(_head_body, num_graphs=num_graphs),
        out_shape=jax.ShapeDtypeStruct((num_graphs, nc), F32),
    )(h, batch2d, fbw, fbb, fw, fb, hbw, hbb, cw, cb)


def _make_agg(n_nodes, nfeat, e_pad):
    epw = e_pad // _NW
    nblk = epw // _EB
    nacc = ((n_nodes + 8) // 8) * 8
    sub_rows = ((nacc // _NSUB + 7) // 8) * 8
    last_rows = nacc - sub_rows * (_NSUB - 1)

    mesh = plsc.VectorSubcoreMesh(core_axis_name="c", subcore_axis_name="s",
                                  num_cores=_NCORES, num_subcores=_NSUB)

    @functools.partial(
        pl.kernel,
        mesh=mesh,
        out_type=jax.ShapeDtypeStruct((_NCORES, nacc, nfeat), F32),
        scratch_types=[
            pltpu.VMEM((_NIDX, 2, _EB), jnp.int32),
            pltpu.VMEM((_NBUF * _EB, nfeat), F32),
            pltpu.VMEM_SHARED((nacc, nfeat), F32),
            pltpu.SemaphoreType.DMA,
            pltpu.SemaphoreType.DMA,
            pltpu.SemaphoreType.DMA,
        ],
    )
    def agg(h_hbm, sd_hbm, out_hbm, sdv, rows, acc, sem_i, sem_g, sem_s):
        c = lax.axis_index("c")
        s = lax.axis_index("s")
        wid = s * _NCORES + c

        for j in range(min(_NIDX - 1, nblk)):
            pltpu.async_copy(sd_hbm.at[wid, j], sdv.at[j], sem_i)

        def zrow(i, carry):
            for j in range(nfeat // 16):
                rows[i, pl.ds(16 * j, 16)] = jnp.zeros((16,), F32)
            return carry
        lax.fori_loop(0, _EB, zrow, 0)
        r0 = s * sub_rows

        def zero_span(nrows):
            for j in range((nrows + _EB - 1) // _EB):
                zr = min(_EB, nrows - j * _EB)
                pltpu.sync_copy(rows.at[pl.ds(0, zr)],
                                acc.at[pl.ds(r0 + j * _EB, zr)])

        @pl.when(s < _NSUB - 1)
        def _():
            zero_span(sub_rows)

        @pl.when(s == _NSUB - 1)
        def _():
            zero_span(last_rows)

        def buf(b):
            return rows.at[pl.ds(lax.rem(b, _NBUF) * _EB, _EB)]

        def sidx(b):
            return sdv.at[lax.rem(b, _NIDX), 0]

        def didx(b):
            return sdv.at[lax.rem(b, _NIDX), 1]

        def wait_idx(b):
            pltpu.make_async_copy(sd_hbm.at[wid, b],
                                  sdv.at[lax.rem(b, _NIDX)], sem_i).wait()

        for j in range(min(_NBUF - 1, nblk)):
            wait_idx(j)
            pltpu.async_copy(h_hbm.at[sidx(j)], buf(j), sem_g)
        plsc.subcore_barrier()

        def block(b, carry):
            @pl.when(b > 0)
            def _():
                pltpu.make_async_copy(buf(b - 1), acc.at[didx(b - 1)],
                                      sem_s).wait()

            @pl.when(b + _NIDX - 1 < nblk)
            def _():
                pltpu.async_copy(sd_hbm.at[wid, b + _NIDX - 1],
                                 sdv.at[lax.rem(b + _NIDX - 1, _NIDX)],
                                 sem_i)

            nxt = b + _NBUF - 1

            @pl.when(nxt < nblk)
            def _():
                wait_idx(nxt)
                pltpu.async_copy(h_hbm.at[sidx(nxt)], buf(nxt), sem_g)

            pltpu.make_async_copy(h_hbm.at[sidx(b)], buf(b), sem_g).wait()
            pltpu.async_copy(buf(b), acc.at[didx(b)], sem_s, add=True)
            return carry
        lax.fori_loop(0, nblk, block, 0)
        pltpu.make_async_copy(buf(nblk - 1), acc.at[didx(nblk - 1)],
                              sem_s).wait()
        plsc.subcore_barrier()

        @pl.when(s < _NSUB - 1)
        def _():
            pltpu.sync_copy(acc.at[pl.ds(r0, sub_rows)],
                            out_hbm.at[c, pl.ds(r0, sub_rows)])

        @pl.when(s == _NSUB - 1)
        def _():
            pltpu.sync_copy(acc.at[pl.ds(r0, last_rows)],
                            out_hbm.at[c, pl.ds(r0, last_rows)])

    return agg


def kernel(x, params, edge_index, batch):
    n, nf = x.shape
    hid = params['conv_W'].shape[1]
    g = 128
    e = edge_index.shape[1]

    e_pad = ((e + _NW * _EB - 1) // (_NW * _EB)) * (_NW * _EB)
    pad = e_pad - e
    nblk = e_pad // (_NW * _EB)
    src = jnp.concatenate([edge_index[0], jnp.zeros((pad,), jnp.int32)])
    dst = jnp.concatenate([edge_index[1], jnp.full((pad,), n, jnp.int32)])
    sd = jnp.stack([src.reshape(_NW, nblk, _EB),
                    dst.reshape(_NW, nblk, _EB)], axis=2)

    r = lambda v: v.reshape(1, -1)
    h = _pre(x, r(params['bn_feat_w']), r(params['bn_feat_b']),
             params['conv_W'], r(params['conv_b']))

    agg = _make_agg(n, hid, e_pad)
    for layer in params['gin']:
        parts = agg(h, sd)
        h = _gin(h, parts, layer['W1'], r(layer['b1']),
                 r(layer['bn_w']), r(layer['bn_b']),
                 layer['W2'], r(layer['b2']))

    out = _head(h, batch.reshape(-1, 1), g,
                r(params['fc_bn_w']), r(params['fc_bn_b']),
                params['fc_W'], r(params['fc_b']),
                r(params['bn_hid_w']), r(params['bn_hid_b']),
                params['cls_W'], r(params['cls_b']))
    return out


# speedup vs baseline: 9.5574x; 1.0682x over previous
import functools

import jax
import jax.numpy as jnp
from jax import lax
from jax.experimental import pallas as pl
from jax.experimental.pallas import tpu as pltpu
from jax.experimental.pallas import tpu_sc as plsc

F32 = jnp.float32

_NCORES = 2
_NSUB = 16
_NW = _NCORES * _NSUB

_EB = 88
_NBUF = 4
_NIDX = 6


def _bn(t, w, b, n_rows, eps=1e-5):
    mu = jnp.mean(t, axis=0, keepdims=True)
    var = jnp.mean((t - mu) ** 2, axis=0, keepdims=True)
    return (t - mu) * lax.rsqrt(var + eps) * w + b


def _pre_body(x_ref, bw_ref, bb_ref, w_ref, b_ref, o_ref):
    x = x_ref[...]
    h = _bn(x, bw_ref[...], bb_ref[...], x.shape[0])
    h = jnp.maximum(jnp.dot(h, w_ref[...], preferred_element_type=F32)
                    + b_ref[...], 0.0)
    o_ref[...] = h


def _pre(x, bw, bb, w, b):
    return pl.pallas_call(
        _pre_body,
        out_shape=jax.ShapeDtypeStruct(x.shape, F32),
    )(x, bw, bb, w, b)


def _gin_body(h_ref, p_ref, w1_ref, b1_ref, bw_ref, bb_ref, w2_ref, b2_ref,
              o_ref):
    n = h_ref.shape[0]
    agg = h_ref[...] + p_ref[0, :n] + p_ref[1, :n]
    t = jnp.dot(agg, w1_ref[...], preferred_element_type=F32) + b1_ref[...]
    t = _bn(t, bw_ref[...], bb_ref[...], t.shape[0])
    t = jnp.maximum(t, 0.0)
    t = jnp.dot(t, w2_ref[...], preferred_element_type=F32) + b2_ref[...]
    o_ref[...] = jnp.maximum(t, 0.0)


def _gin(h, parts, w1, b1, bw, bb, w2, b2):
    return pl.pallas_call(
        _gin_body,
        out_shape=jax.ShapeDtypeStruct(h.shape, F32),
    )(h, parts, w1, b1, bw, bb, w2, b2)


def _head_body(h_ref, batch_ref, fbw_ref, fbb_ref, fw_ref, fb_ref,
               hbw_ref, hbb_ref, cw_ref, cb_ref, o_ref, *, num_graphs):
    h = h_ref[...]
    n = h.shape[0]
    onehot = (batch_ref[...] ==
              lax.broadcasted_iota(jnp.int32, (n, num_graphs), 1)).astype(F32)
    pooled = lax.dot_general(onehot, h, (((0,), (0,)), ((), ())),
                             preferred_element_type=F32)
    z = _bn(pooled, fbw_ref[...], fbb_ref[...], num_graphs)
    z = jnp.maximum(jnp.dot(z, fw_ref[...], preferred_element_type=F32)
                    + fb_ref[...], 0.0)
    z = _bn(z, hbw_ref[...], hbb_ref[...], num_graphs)
    z = jnp.dot(z, cw_ref[...], preferred_element_type=F32) + cb_ref[...]
    m = jnp.max(z, axis=-1, keepdims=True)
    lse = jnp.log(jnp.sum(jnp.exp(z - m), axis=-1, keepdims=True)) + m
    o_ref[...] = z - lse


def _head(h, batch2d, num_graphs, fbw, fbb, fw, fb, hbw, hbb, cw, cb):
    nc = cw.shape[1]
    return pl.pallas_call(
        functools.partial(_head_body, num_graphs=num_graphs),
        out_shape=jax.ShapeDtypeStruct((num_graphs, nc), F32),
    )(h, batch2d, fbw, fbb, fw, fb, hbw, hbb, cw, cb)


def _make_agg(n_nodes, nfeat, e_pad, frac0):
    nb_tot = e_pad // _EB
    nb0 = int(round(frac0 * nb_tot / _NSUB))
    nb1 = nb_tot // _NSUB - nb0
    nacc = ((n_nodes + 8) // 8) * 8
    sub_rows = ((nacc // _NSUB + 7) // 8) * 8
    last_rows = nacc - sub_rows * (_NSUB - 1)

    mesh = plsc.VectorSubcoreMesh(core_axis_name="c", subcore_axis_name="s",
                                  num_cores=_NCORES, num_subcores=_NSUB)

    @functools.partial(
        pl.kernel,
        mesh=mesh,
        out_type=jax.ShapeDtypeStruct((_NCORES, nacc, nfeat), F32),
        scratch_types=[
            pltpu.VMEM((_NIDX, 2, _EB), jnp.int32),
            pltpu.VMEM((_NBUF * _EB, nfeat), F32),
            pltpu.VMEM_SHARED((nacc, nfeat), F32),
            pltpu.SemaphoreType.DMA,
            pltpu.SemaphoreType.DMA,
            pltpu.SemaphoreType.DMA,
        ],
    )
    def agg(h_hbm, sd_hbm, out_hbm, sdv, rows, acc, sem_i, sem_g, sem_s):
        c = lax.axis_index("c")
        s = lax.axis_index("s")
        nblk = jnp.where(c == 0, nb0, nb1)
        base = jnp.where(c == 0, s * nb0, _NSUB * nb0 + s * nb1)

        for j in range(_NIDX - 1):
            pltpu.async_copy(sd_hbm.at[base + j], sdv.at[j], sem_i)

        def zrow(i, carry):
            for j in range(nfeat // 16):
                rows[i, pl.ds(16 * j, 16)] = jnp.zeros((16,), F32)
            return carry
        lax.fori_loop(0, _EB, zrow, 0)
        r0 = s * sub_rows

        def zero_span(nrows):
            for j in range((nrows + _EB - 1) // _EB):
                zr = min(_EB, nrows - j * _EB)
                pltpu.sync_copy(rows.at[pl.ds(0, zr)],
                                acc.at[pl.ds(r0 + j * _EB, zr)])

        @pl.when(s < _NSUB - 1)
        def _():
            zero_span(sub_rows)

        @pl.when(s == _NSUB - 1)
        def _():
            zero_span(last_rows)

        def buf(b):
            return rows.at[pl.ds(lax.rem(b, _NBUF) * _EB, _EB)]

        def sidx(b):
            return sdv.at[lax.rem(b, _NIDX), 0]

        def didx(b):
            return sdv.at[lax.rem(b, _NIDX), 1]

        def wait_idx(b):
            pltpu.make_async_copy(sd_hbm.at[base + b],
                                  sdv.at[lax.rem(b, _NIDX)], sem_i).wait()

        for j in range(_NBUF - 1):
            wait_idx(j)
            pltpu.async_copy(h_hbm.at[sidx(j)], buf(j), sem_g)
        plsc.subcore_barrier()

        def block(b, carry):
            @pl.when(b > 0)
            def _():
                pltpu.make_async_copy(buf(b - 1), acc.at[didx(b - 1)],
                                      sem_s).wait()

            @pl.when(b + _NIDX - 1 < nblk)
            def _():
                pltpu.async_copy(sd_hbm.at[base + b + _NIDX - 1],
                                 sdv.at[lax.rem(b + _NIDX - 1, _NIDX)],
                                 sem_i)

            nxt = b + _NBUF - 1

            @pl.when(nxt < nblk)
            def _():
                wait_idx(nxt)
                pltpu.async_copy(h_hbm.at[sidx(nxt)], buf(nxt), sem_g)

            pltpu.make_async_copy(h_hbm.at[sidx(b)], buf(b), sem_g).wait()
            pltpu.async_copy(buf(b), acc.at[didx(b)], sem_s, add=True)
            return carry
        lax.fori_loop(0, nblk, block, 0)
        pltpu.make_async_copy(buf(nblk - 1), acc.at[didx(nblk - 1)],
                              sem_s).wait()
        plsc.subcore_barrier()

        @pl.when(s < _NSUB - 1)
        def _():
            pltpu.sync_copy(acc.at[pl.ds(r0, sub_rows)],
                            out_hbm.at[c, pl.ds(r0, sub_rows)])

        @pl.when(s == _NSUB - 1)
        def _():
            pltpu.sync_copy(acc.at[pl.ds(r0, last_rows)],
                            out_hbm.at[c, pl.ds(r0, last_rows)])

    return agg


def kernel(x, params, edge_index, batch):
    n, nf = x.shape
    hid = params['conv_W'].shape[1]
    g = 128
    e = edge_index.shape[1]

    e_pad = ((e + _NW * _EB - 1) // (_NW * _EB)) * (_NW * _EB)
    pad = e_pad - e
    nblk = e_pad // (_NW * _EB)
    src = jnp.concatenate([edge_index[0], jnp.zeros((pad,), jnp.int32)])
    dst = jnp.concatenate([edge_index[1], jnp.full((pad,), n, jnp.int32)])
    sd = jnp.stack([src.reshape(-1, _EB), dst.reshape(-1, _EB)], axis=1)

    r = lambda v: v.reshape(1, -1)
    h = _pre(x, r(params['bn_feat_w']), r(params['bn_feat_b']),
             params['conv_W'], r(params['conv_b']))

    agg = _make_agg(n, hid, e_pad, 0.61)
    for layer in params['gin']:
        parts = agg(h, sd)
        h = _gin(h, parts, layer['W1'], r(layer['b1']),
                 r(layer['bn_w']), r(layer['bn_b']),
                 layer['W2'], r(layer['b2']))

    out = _head(h, batch.reshape(-1, 1), g,
                r(params['fc_bn_w']), r(params['fc_bn_b']),
                params['fc_W'], r(params['fc_b']),
                r(params['bn_hid_w']), r(params['bn_hid_b']),
                params['cls_W'], r(params['cls_b']))
    return out
